# baseline (speedup 1.0000x reference)
import functools

import jax
import jax.numpy as jnp
from jax import lax
from jax.experimental import pallas as pl
from jax.experimental.pallas import tpu as pltpu
from jax.experimental.pallas import tpu_sc as plsc

N_CLS = 1000
N_CTX = 16
CTX_DIM = 512
BATCH = 4096
SUF_LEN = 60
TOT_LEN = 1 + N_CTX + SUF_LEN

_D = CTX_DIM
_CTX_W = N_CTX * _D

_NC = 2
_NS = 16
_NW = _NC * _NS
_BPW = BATCH // _NW
_CH = 8
_NCHUNK = _BPW // _CH


def _sc_gather_kernel():
    mesh = plsc.VectorSubcoreMesh(core_axis_name="c", subcore_axis_name="s")

    @functools.partial(
        pl.kernel,
        mesh=mesh,
        out_type=jax.ShapeDtypeStruct((BATCH, _CTX_W), jnp.float32),
        scratch_types=[
            pltpu.VMEM((_BPW,), jnp.int32),
            pltpu.VMEM((_CH, _CTX_W), jnp.float32),
            pltpu.SemaphoreType.DMA,
        ],
    )
    def k(label_hbm, ctx_hbm, out_hbm, idx_v, buf, sem):
        wid = lax.axis_index("s") * _NC + lax.axis_index("c")
        base = wid * _BPW

        pltpu.sync_copy(label_hbm.at[pl.ds(base, _BPW)], idx_v.at[...])

        def body(c, carry):
            pltpu.async_copy(
                ctx_hbm.at[idx_v.at[pl.ds(c * _CH, _CH)]], buf, sem).wait()
            pltpu.sync_copy(buf.at[...], out_hbm.at[pl.ds(base + c * _CH, _CH)])
            return carry

        lax.fori_loop(0, _NCHUNK, body, 0)

    return k


_BB = 128


def _tc_suffix(suf_ref, out_ref):
    out_ref[...] = suf_ref[...]


def _tc_suffix_call(suffix_t):
    return pl.pallas_call(
        _tc_suffix,
        grid=(2, SUF_LEN),
        in_specs=[pl.BlockSpec((1, BATCH // 2, _D), lambda h, s: (s, h, 0))],
        out_specs=pl.BlockSpec((1, BATCH // 2, _D), lambda h, s: (1 + N_CTX + s, h, 0)),
        out_shape=jax.ShapeDtypeStruct((TOT_LEN, BATCH, _D), jnp.float32),
    )(suffix_t)


def _tc_head(out1_ref, pref_ref, gath_ref, out_ref):
    out_ref[0, :, :] = pref_ref[:, 0, :]
    for t in range(N_CTX):
        out_ref[1 + t, :, :] = gath_ref[:, t * _D:(t + 1) * _D]


def _tc_head_call(out1, prefix, gathered):
    return pl.pallas_call(
        _tc_head,
        grid=(BATCH // _BB,),
        in_specs=[
            pl.BlockSpec(memory_space=pl.ANY),
            pl.BlockSpec((_BB, 1, _D), lambda i: (i, 0, 0)),
            pl.BlockSpec((_BB, _CTX_W), lambda i: (i, 0)),
        ],
        out_specs=pl.BlockSpec((1 + N_CTX, _BB, _D), lambda i: (0, i, 0)),
        out_shape=jax.ShapeDtypeStruct((TOT_LEN, BATCH, _D), jnp.float32),
        input_output_aliases={0: 0},
    )(out1, prefix, gathered)


def kernel(label, prefix, suffix, ctx):
    label32 = label.astype(jnp.int32).reshape(BATCH)
    ctx2 = ctx.reshape(N_CLS, _CTX_W)
    suffix_t = suffix.transpose(1, 0, 2)
    gathered = _sc_gather_kernel()(label32, ctx2)
    out1 = _tc_suffix_call(suffix_t)
    out_t = _tc_head_call(out1, prefix, gathered)
    return out_t.transpose(1, 0, 2)


# speedup vs baseline: 1.0065x; 1.0065x over previous
import functools

import jax
import jax.numpy as jnp
from jax import lax
from jax.experimental import pallas as pl
from jax.experimental.pallas import tpu as pltpu
from jax.experimental.pallas import tpu_sc as plsc

N_CLS = 1000
N_CTX = 16
CTX_DIM = 512
BATCH = 4096
SUF_LEN = 60
TOT_LEN = 1 + N_CTX + SUF_LEN

_D = CTX_DIM
_CTX_W = N_CTX * _D

_NC = 2
_NS = 16
_NW = _NC * _NS
_BPW = BATCH // _NW
_CH = 8
_NCHUNK = _BPW // _CH


def _sc_gather_kernel():
    mesh = plsc.VectorSubcoreMesh(core_axis_name="c", subcore_axis_name="s")

    @functools.partial(
        pl.kernel,
        mesh=mesh,
        out_type=jax.ShapeDtypeStruct((BATCH, _CTX_W), jnp.float32),
        scratch_types=[
            pltpu.VMEM((_BPW,), jnp.int32),
            pltpu.VMEM((_CH, _CTX_W), jnp.float32),
            pltpu.SemaphoreType.DMA,
        ],
    )
    def k(label_hbm, ctx_hbm, out_hbm, idx_v, buf, sem):
        wid = lax.axis_index("s") * _NC + lax.axis_index("c")
        base = wid * _BPW

        pltpu.sync_copy(label_hbm.at[pl.ds(base, _BPW)], idx_v.at[...])

        def body(c, carry):
            pltpu.async_copy(
                ctx_hbm.at[idx_v.at[pl.ds(c * _CH, _CH)]], buf, sem).wait()
            pltpu.sync_copy(buf.at[...], out_hbm.at[pl.ds(base + c * _CH, _CH)])
            return carry

        lax.fori_loop(0, _NCHUNK, body, 0)

    return k


_BB = 128


def _tc_suffix(suf_ref, out_ref):
    out_ref[...] = suf_ref[...]


def _tc_suffix_call(suffix_t):
    return pl.pallas_call(
        _tc_suffix,
        grid=(SUF_LEN,),
        in_specs=[pl.BlockSpec((1, BATCH, _D), lambda s: (s, 0, 0))],
        out_specs=pl.BlockSpec((1, BATCH, _D), lambda s: (1 + N_CTX + s, 0, 0)),
        out_shape=jax.ShapeDtypeStruct((TOT_LEN, BATCH, _D), jnp.float32),
    )(suffix_t)


def _tc_head(out1_ref, pref_ref, gath_ref, out_ref):
    out_ref[0, :, :] = pref_ref[:, 0, :]
    for t in range(N_CTX):
        out_ref[1 + t, :, :] = gath_ref[:, t * _D:(t + 1) * _D]


def _tc_head_call(out1, prefix, gathered):
    return pl.pallas_call(
        _tc_head,
        grid=(BATCH // _BB,),
        in_specs=[
            pl.BlockSpec(memory_space=pl.ANY),
            pl.BlockSpec((_BB, 1, _D), lambda i: (i, 0, 0)),
            pl.BlockSpec((_BB, _CTX_W), lambda i: (i, 0)),
        ],
        out_specs=pl.BlockSpec((1 + N_CTX, _BB, _D), lambda i: (0, i, 0)),
        out_shape=jax.ShapeDtypeStruct((TOT_LEN, BATCH, _D), jnp.float32),
        input_output_aliases={0: 0},
    )(out1, prefix, gathered)


def kernel(label, prefix, suffix, ctx):
    label32 = label.astype(jnp.int32).reshape(BATCH)
    suffix_t = suffix.transpose(1, 0, 2)
    out1 = _tc_suffix_call(suffix_t)
    ctx2 = ctx.reshape(N_CLS, _CTX_W)
    gathered = _sc_gather_kernel()(label32, ctx2)
    out_t = _tc_head_call(out1, prefix, gathered)
    return out_t.transpose(1, 0, 2)
